# baseline (speedup 1.0000x reference)
import jax
import jax.numpy as jnp
from jax import lax
from jax.experimental import pallas as pl
from jax.experimental.pallas import tpu as pltpu
from jax.experimental.pallas import tpu_sc as plsc

DIM = 32
BATCH = 16384

NC = 2
NS = 16
NW = NC * NS
BPW = BATCH // NW
CHUNK = 128
NCHUNK = BPW // CHUNK
GPC = CHUNK // 16


def _mure_body(u_idx_hbm, r_idx_hbm, v_idx_hbm, E4_hbm, Wu4_hbm, rv4_hbm,
               bs_hbm, bo_hbm, out_hbm,
               idx_u, idx_r, idx_v, idx4_u, idx4_r, idx4_v,
               u_rows, v_rows, ru_rows, rr_rows,
               bs_v, bo_v, out_v, sem):
    wid = lax.axis_index("s") * NC + lax.axis_index("c")
    base = wid * BPW
    lane = lax.iota(jnp.int32, 16)

    for j in range(NCHUNK):
        sl = pl.ds(base + j * CHUNK, CHUNK)
        pltpu.sync_copy(u_idx_hbm.at[sl], idx_u.at[j])
        pltpu.sync_copy(r_idx_hbm.at[sl], idx_r.at[j])
        pltpu.sync_copy(v_idx_hbm.at[sl], idx_v.at[j])

    def shift(i, carry):
        j = i // GPC
        sl = pl.ds((i % GPC) * 16, 16)
        idx4_u[j, sl] = lax.shift_right_logical(idx_u[j, sl], 2)
        idx4_r[j, sl] = lax.shift_right_logical(idx_r[j, sl], 2)
        idx4_v[j, sl] = lax.shift_right_logical(idx_v[j, sl], 2)
        return carry

    lax.fori_loop(0, NCHUNK * GPC, shift, 0)

    for j in range(NCHUNK):
        csl = pl.ds(j * CHUNK, CHUNK)
        cps = [
            pltpu.async_copy(E4_hbm.at[idx4_u.at[j]], u_rows, sem),
            pltpu.async_copy(E4_hbm.at[idx4_v.at[j]], v_rows, sem),
            pltpu.async_copy(Wu4_hbm.at[idx4_r.at[j]], ru_rows, sem),
            pltpu.async_copy(rv4_hbm.at[idx4_r.at[j]], rr_rows, sem),
            pltpu.async_copy(bs_hbm.at[idx_u.at[j]], bs_v.at[csl], sem),
            pltpu.async_copy(bo_hbm.at[idx_v.at[j]], bo_v.at[csl], sem),
        ]
        for cp in cps:
            cp.wait()

        def group(g, carry):
            sl = pl.ds(g * 16, 16)
            rows16 = g * 16 + lane
            off_u = (idx_u[j, sl] & 3) * DIM
            off_r = (idx_r[j, sl] & 3) * DIM
            off_v = (idx_v[j, sl] & 3) * DIM
            acc = jnp.zeros((16,), jnp.float32)
            for d in range(DIM):
                uu = plsc.load_gather(u_rows, [rows16, off_u + d])
                ru = plsc.load_gather(ru_rows, [rows16, off_r + d])
                vv = plsc.load_gather(v_rows, [rows16, off_v + d])
                rr = plsc.load_gather(rr_rows, [rows16, off_r + d])
                t = uu * ru - vv - rr
                acc = acc + t * t
            osl = pl.ds(j * CHUNK + g * 16, 16)
            out_v[osl] = bs_v[osl] + bo_v[osl] - acc
            return carry

        lax.fori_loop(0, GPC, group, 0)

    pltpu.sync_copy(out_v, out_hbm.at[pl.ds(base, BPW)])


@jax.jit
def _mure_sc(u_idx, r_idx, v_idx, E4, Wu4, rv4, bs, bo):
    mesh = plsc.VectorSubcoreMesh(core_axis_name="c", subcore_axis_name="s")
    return pl.kernel(
        _mure_body,
        mesh=mesh,
        compiler_params=pltpu.CompilerParams(
            needs_layout_passes=False, use_tc_tiling_on_sc=False),
        out_type=jax.ShapeDtypeStruct((BATCH,), jnp.float32),
        scratch_types=[
            pltpu.VMEM((NCHUNK, CHUNK), jnp.int32),
            pltpu.VMEM((NCHUNK, CHUNK), jnp.int32),
            pltpu.VMEM((NCHUNK, CHUNK), jnp.int32),
            pltpu.VMEM((NCHUNK, CHUNK), jnp.int32),
            pltpu.VMEM((NCHUNK, CHUNK), jnp.int32),
            pltpu.VMEM((NCHUNK, CHUNK), jnp.int32),
            pltpu.VMEM((CHUNK, 128), jnp.float32),
            pltpu.VMEM((CHUNK, 128), jnp.float32),
            pltpu.VMEM((CHUNK, 128), jnp.float32),
            pltpu.VMEM((CHUNK, 128), jnp.float32),
            pltpu.VMEM((BPW,), jnp.float32),
            pltpu.VMEM((BPW,), jnp.float32),
            pltpu.VMEM((BPW,), jnp.float32),
            pltpu.SemaphoreType.DMA,
        ],
    )(u_idx, r_idx, v_idx, E4, Wu4, rv4, bs, bo)


def kernel(u_idx, r_idx, v_idx, E, Wu, rv, bs, bo):
    E4 = E.reshape(-1, 128)
    Wu4 = Wu.reshape(-1, 128)
    rv4 = rv.reshape(-1, 128)
    return _mure_sc(u_idx, r_idx, v_idx, E4, Wu4, rv4, bs, bo)


# speedup vs baseline: 1.0007x; 1.0007x over previous
import jax
import jax.numpy as jnp
from jax import lax
from jax.experimental import pallas as pl
from jax.experimental.pallas import tpu as pltpu
from jax.experimental.pallas import tpu_sc as plsc

DIM = 32
BATCH = 16384

NC = 2
NS = 16
NW = NC * NS
BPW = BATCH // NW
CHUNK = 128
NCHUNK = BPW // CHUNK
GPC = CHUNK // 16


def _mure_body(u_idx_hbm, r_idx_hbm, v_idx_hbm, E4_hbm, Wu4_hbm, rv4_hbm,
               bs_hbm, bo_hbm, out_hbm,
               idx_u, idx_r, idx_v, idx4_u, idx4_r, idx4_v,
               u_rows, v_rows, ru_rows, rr_rows,
               bs_v, bo_v, out_v, sem):
    wid = lax.axis_index("s") * NC + lax.axis_index("c")
    base = wid * BPW
    lane = lax.iota(jnp.int32, 16)

    for j in range(NCHUNK):
        sl = pl.ds(base + j * CHUNK, CHUNK)
        pltpu.sync_copy(u_idx_hbm.at[sl], idx_u.at[j])
        pltpu.sync_copy(r_idx_hbm.at[sl], idx_r.at[j])
        pltpu.sync_copy(v_idx_hbm.at[sl], idx_v.at[j])

    def shift(i, carry):
        j = i // GPC
        sl = pl.ds((i % GPC) * 16, 16)
        idx4_u[j, sl] = lax.shift_right_logical(idx_u[j, sl], 2)
        idx4_r[j, sl] = lax.shift_right_logical(idx_r[j, sl], 2)
        idx4_v[j, sl] = lax.shift_right_logical(idx_v[j, sl], 2)
        return carry

    lax.fori_loop(0, NCHUNK * GPC, shift, 0)

    for j in range(NCHUNK):
        csl = pl.ds(j * CHUNK, CHUNK)
        cps = [
            pltpu.async_copy(E4_hbm.at[idx4_u.at[j]], u_rows, sem),
            pltpu.async_copy(E4_hbm.at[idx4_v.at[j]], v_rows, sem),
            pltpu.async_copy(Wu4_hbm.at[idx4_r.at[j]], ru_rows, sem),
            pltpu.async_copy(rv4_hbm.at[idx4_r.at[j]], rr_rows, sem),
            pltpu.async_copy(bs_hbm.at[idx_u.at[j]], bs_v.at[csl], sem),
            pltpu.async_copy(bo_hbm.at[idx_v.at[j]], bo_v.at[csl], sem),
        ]
        for cp in cps:
            cp.wait()

        def group(g, carry):
            sl = pl.ds(g * 16, 16)
            rows16 = g * 16 + lane
            off_u = (idx_u[j, sl] & 3) * DIM
            off_r = (idx_r[j, sl] & 3) * DIM
            off_v = (idx_v[j, sl] & 3) * DIM
            acc = jnp.zeros((16,), jnp.float32)
            for d in range(DIM):
                uu = plsc.load_gather(u_rows, [rows16, off_u + d])
                ru = plsc.load_gather(ru_rows, [rows16, off_r + d])
                vv = plsc.load_gather(v_rows, [rows16, off_v + d])
                rr = plsc.load_gather(rr_rows, [rows16, off_r + d])
                t = uu * ru - vv - rr
                acc = acc + t * t
            osl = pl.ds(j * CHUNK + g * 16, 16)
            out_v[osl] = bs_v[osl] + bo_v[osl] - acc
            return carry

        lax.fori_loop(0, GPC, group, 0)

    pltpu.sync_copy(out_v, out_hbm.at[pl.ds(base, BPW)])


@jax.jit
def _mure_sc(u_idx, r_idx, v_idx, E4, Wu4, rv4, bs, bo):
    mesh = plsc.VectorSubcoreMesh(core_axis_name="c", subcore_axis_name="s")
    return pl.kernel(
        _mure_body,
        mesh=mesh,
        compiler_params=pltpu.CompilerParams(
            needs_layout_passes=False, use_tc_tiling_on_sc=True),
        out_type=jax.ShapeDtypeStruct((BATCH,), jnp.float32),
        scratch_types=[
            pltpu.VMEM((NCHUNK, CHUNK), jnp.int32),
            pltpu.VMEM((NCHUNK, CHUNK), jnp.int32),
            pltpu.VMEM((NCHUNK, CHUNK), jnp.int32),
            pltpu.VMEM((NCHUNK, CHUNK), jnp.int32),
            pltpu.VMEM((NCHUNK, CHUNK), jnp.int32),
            pltpu.VMEM((NCHUNK, CHUNK), jnp.int32),
            pltpu.VMEM((CHUNK, 128), jnp.float32),
            pltpu.VMEM((CHUNK, 128), jnp.float32),
            pltpu.VMEM((CHUNK, 128), jnp.float32),
            pltpu.VMEM((CHUNK, 128), jnp.float32),
            pltpu.VMEM((BPW,), jnp.float32),
            pltpu.VMEM((BPW,), jnp.float32),
            pltpu.VMEM((BPW,), jnp.float32),
            pltpu.SemaphoreType.DMA,
        ],
    )(u_idx, r_idx, v_idx, E4, Wu4, rv4, bs, bo)


def kernel(u_idx, r_idx, v_idx, E, Wu, rv, bs, bo):
    E4 = E.reshape(-1, 128)
    Wu4 = Wu.reshape(-1, 128)
    rv4 = rv.reshape(-1, 128)
    return _mure_sc(u_idx, r_idx, v_idx, E4, Wu4, rv4, bs, bo)


# speedup vs baseline: 1.1558x; 1.1549x over previous
import jax
import jax.numpy as jnp
from jax import lax
from jax.experimental import pallas as pl
from jax.experimental.pallas import tpu as pltpu
from jax.experimental.pallas import tpu_sc as plsc

DIM = 32
BATCH = 16384

NC = 2
NS = 16
NW = NC * NS
BPW = BATCH // NW
CHUNK = 128
NCHUNK = BPW // CHUNK
GPC = CHUNK // 16


def _detile_body(x0, x1, x2, x3, out_ref):
    out_ref[...] = jnp.concatenate(
        [x0[...].T, x1[...].T, x2[...].T, x3[...].T], axis=1)


def _detile(et, bn):
    n = et.shape[1]
    qb = bn // 4
    last = (n + qb - 1) // qb - 1
    specs = [
        pl.BlockSpec((DIM, qb), lambda i, c=c: (0, jnp.minimum(4 * i + c, last)))
        for c in range(4)
    ]
    grid = (n + bn - 1) // bn
    return pl.pallas_call(
        _detile_body,
        grid=(grid,),
        in_specs=specs,
        out_specs=pl.BlockSpec((qb, 128), lambda i: (i, 0)),
        out_shape=jax.ShapeDtypeStruct((grid * qb, 128), jnp.float32),
    )(et, et, et, et)


def _mure_body(u_idx_hbm, r_idx_hbm, v_idx_hbm, E4_hbm, Wu4_hbm, rv4_hbm,
               bs_hbm, bo_hbm, out_hbm,
               idx_u, idx_r, idx_v, idx4_u, idx4_r, idx4_v,
               u_rows, v_rows, ru_rows, rr_rows,
               bs_v, bo_v, out_v, sem):
    wid = lax.axis_index("s") * NC + lax.axis_index("c")
    base = wid * BPW
    lane = lax.iota(jnp.int32, 16)

    for j in range(NCHUNK):
        sl = pl.ds(base + j * CHUNK, CHUNK)
        pltpu.sync_copy(u_idx_hbm.at[sl], idx_u.at[j])
        pltpu.sync_copy(r_idx_hbm.at[sl], idx_r.at[j])
        pltpu.sync_copy(v_idx_hbm.at[sl], idx_v.at[j])

    def shift(i, carry):
        j = i // GPC
        sl = pl.ds((i % GPC) * 16, 16)
        u = idx_u[j, sl]
        r = idx_r[j, sl]
        v = idx_v[j, sl]
        idx4_u[j, sl] = ((u >> 11) << 9) | (u & 511)
        idx4_r[j, sl] = ((r >> 9) << 7) | (r & 127)
        idx4_v[j, sl] = ((v >> 11) << 9) | (v & 511)
        return carry

    lax.fori_loop(0, NCHUNK * GPC, shift, 0)

    for j in range(NCHUNK):
        csl = pl.ds(j * CHUNK, CHUNK)
        cps = [
            pltpu.async_copy(E4_hbm.at[idx4_u.at[j]], u_rows, sem),
            pltpu.async_copy(E4_hbm.at[idx4_v.at[j]], v_rows, sem),
            pltpu.async_copy(Wu4_hbm.at[idx4_r.at[j]], ru_rows, sem),
            pltpu.async_copy(rv4_hbm.at[idx4_r.at[j]], rr_rows, sem),
            pltpu.async_copy(bs_hbm.at[idx_u.at[j]], bs_v.at[csl], sem),
            pltpu.async_copy(bo_hbm.at[idx_v.at[j]], bo_v.at[csl], sem),
        ]
        for cp in cps:
            cp.wait()

        def group(g, carry):
            sl = pl.ds(g * 16, 16)
            rows16 = g * 16 + lane
            off_u = ((idx_u[j, sl] >> 9) & 3) * DIM
            off_r = ((idx_r[j, sl] >> 7) & 3) * DIM
            off_v = ((idx_v[j, sl] >> 9) & 3) * DIM
            acc = jnp.zeros((16,), jnp.float32)
            for d in range(DIM):
                uu = plsc.load_gather(u_rows, [rows16, off_u + d])
                ru = plsc.load_gather(ru_rows, [rows16, off_r + d])
                vv = plsc.load_gather(v_rows, [rows16, off_v + d])
                rr = plsc.load_gather(rr_rows, [rows16, off_r + d])
                t = uu * ru - vv - rr
                acc = acc + t * t
            osl = pl.ds(j * CHUNK + g * 16, 16)
            out_v[osl] = bs_v[osl] + bo_v[osl] - acc
            return carry

        lax.fori_loop(0, GPC, group, 0)

    pltpu.sync_copy(out_v, out_hbm.at[pl.ds(base, BPW)])


@jax.jit
def _mure_sc(u_idx, r_idx, v_idx, E4, Wu4, rv4, bs, bo):
    mesh = plsc.VectorSubcoreMesh(core_axis_name="c", subcore_axis_name="s")
    return pl.kernel(
        _mure_body,
        mesh=mesh,
        compiler_params=pltpu.CompilerParams(
            needs_layout_passes=False, use_tc_tiling_on_sc=True),
        out_type=jax.ShapeDtypeStruct((BATCH,), jnp.float32),
        scratch_types=[
            pltpu.VMEM((NCHUNK, CHUNK), jnp.int32),
            pltpu.VMEM((NCHUNK, CHUNK), jnp.int32),
            pltpu.VMEM((NCHUNK, CHUNK), jnp.int32),
            pltpu.VMEM((NCHUNK, CHUNK), jnp.int32),
            pltpu.VMEM((NCHUNK, CHUNK), jnp.int32),
            pltpu.VMEM((NCHUNK, CHUNK), jnp.int32),
            pltpu.VMEM((CHUNK, 128), jnp.float32),
            pltpu.VMEM((CHUNK, 128), jnp.float32),
            pltpu.VMEM((CHUNK, 128), jnp.float32),
            pltpu.VMEM((CHUNK, 128), jnp.float32),
            pltpu.VMEM((BPW,), jnp.float32),
            pltpu.VMEM((BPW,), jnp.float32),
            pltpu.VMEM((BPW,), jnp.float32),
            pltpu.SemaphoreType.DMA,
        ],
    )(u_idx, r_idx, v_idx, E4, Wu4, rv4, bs, bo)


def kernel(u_idx, r_idx, v_idx, E, Wu, rv, bs, bo):
    E4 = _detile(E.T, 2048)
    Wu4 = _detile(Wu.T, 512)
    rv4 = _detile(rv.T, 512)
    return _mure_sc(u_idx, r_idx, v_idx, E4, Wu4, rv4, bs, bo)


# speedup vs baseline: 3.5178x; 3.0437x over previous
import jax
import jax.numpy as jnp
from jax import lax
from jax.experimental import pallas as pl
from jax.experimental.pallas import tpu as pltpu
from jax.experimental.pallas import tpu_sc as plsc

DIM = 32
BATCH = 16384

NC = 2
NS = 16
NW = NC * NS
BPW = BATCH // NW
CHUNK = 128
NCHUNK = BPW // CHUNK
GPC = CHUNK // 16


def _detile_body(x0, x1, x2, x3, out_ref):
    q = jnp.concatenate([x0[...], x1[...], x2[...], x3[...]], axis=0)
    out_ref[...] = q.T


def _detile(et, bn):
    n = et.shape[1]
    qb = bn // 4
    last = (n + qb - 1) // qb - 1
    specs = [
        pl.BlockSpec((DIM, qb), lambda i, c=c: (0, jnp.minimum(4 * i + c, last)))
        for c in range(4)
    ]
    grid = (n + bn - 1) // bn
    return pl.pallas_call(
        _detile_body,
        grid=(grid,),
        in_specs=specs,
        out_specs=pl.BlockSpec((qb, 128), lambda i: (i, 0)),
        out_shape=jax.ShapeDtypeStruct((grid * qb, 128), jnp.float32),
    )(et, et, et, et)


def _mure_body(u_idx_hbm, r_idx_hbm, v_idx_hbm, E4_hbm, Wu4_hbm, rv4_hbm,
               bs_hbm, bo_hbm, out_hbm,
               idx_u, idx_r, idx_v, idx4_u, idx4_r, idx4_v,
               u_rows, v_rows, ru_rows, rr_rows,
               bs_v, bo_v, out_v, sem):
    wid = lax.axis_index("s") * NC + lax.axis_index("c")
    base = wid * BPW
    lane = lax.iota(jnp.int32, 16)

    for j in range(NCHUNK):
        sl = pl.ds(base + j * CHUNK, CHUNK)
        pltpu.sync_copy(u_idx_hbm.at[sl], idx_u.at[j])
        pltpu.sync_copy(r_idx_hbm.at[sl], idx_r.at[j])
        pltpu.sync_copy(v_idx_hbm.at[sl], idx_v.at[j])

    def shift(i, carry):
        j = i // GPC
        sl = pl.ds((i % GPC) * 16, 16)
        u = idx_u[j, sl]
        r = idx_r[j, sl]
        v = idx_v[j, sl]
        idx4_u[j, sl] = ((u >> 16) << 14) | (u & 16383)
        idx4_r[j, sl] = ((r >> 9) << 7) | (r & 127)
        idx4_v[j, sl] = ((v >> 16) << 14) | (v & 16383)
        return carry

    lax.fori_loop(0, NCHUNK * GPC, shift, 0)

    for j in range(NCHUNK):
        csl = pl.ds(j * CHUNK, CHUNK)
        cps = [
            pltpu.async_copy(E4_hbm.at[idx4_u.at[j]], u_rows, sem),
            pltpu.async_copy(E4_hbm.at[idx4_v.at[j]], v_rows, sem),
            pltpu.async_copy(Wu4_hbm.at[idx4_r.at[j]], ru_rows, sem),
            pltpu.async_copy(rv4_hbm.at[idx4_r.at[j]], rr_rows, sem),
            pltpu.async_copy(bs_hbm.at[idx_u.at[j]], bs_v.at[csl], sem),
            pltpu.async_copy(bo_hbm.at[idx_v.at[j]], bo_v.at[csl], sem),
        ]
        for cp in cps:
            cp.wait()

        def group(g, carry):
            sl = pl.ds(g * 16, 16)
            rows16 = g * 16 + lane
            off_u = ((idx_u[j, sl] >> 14) & 3) * DIM
            off_r = ((idx_r[j, sl] >> 7) & 3) * DIM
            off_v = ((idx_v[j, sl] >> 14) & 3) * DIM
            acc = jnp.zeros((16,), jnp.float32)
            for d in range(DIM):
                uu = plsc.load_gather(u_rows, [rows16, off_u + d])
                ru = plsc.load_gather(ru_rows, [rows16, off_r + d])
                vv = plsc.load_gather(v_rows, [rows16, off_v + d])
                rr = plsc.load_gather(rr_rows, [rows16, off_r + d])
                t = uu * ru - vv - rr
                acc = acc + t * t
            osl = pl.ds(j * CHUNK + g * 16, 16)
            out_v[osl] = bs_v[osl] + bo_v[osl] - acc
            return carry

        lax.fori_loop(0, GPC, group, 0)

    pltpu.sync_copy(out_v, out_hbm.at[pl.ds(base, BPW)])


@jax.jit
def _mure_sc(u_idx, r_idx, v_idx, E4, Wu4, rv4, bs, bo):
    mesh = plsc.VectorSubcoreMesh(core_axis_name="c", subcore_axis_name="s")
    return pl.kernel(
        _mure_body,
        mesh=mesh,
        compiler_params=pltpu.CompilerParams(
            needs_layout_passes=False, use_tc_tiling_on_sc=True),
        out_type=jax.ShapeDtypeStruct((BATCH,), jnp.float32),
        scratch_types=[
            pltpu.VMEM((NCHUNK, CHUNK), jnp.int32),
            pltpu.VMEM((NCHUNK, CHUNK), jnp.int32),
            pltpu.VMEM((NCHUNK, CHUNK), jnp.int32),
            pltpu.VMEM((NCHUNK, CHUNK), jnp.int32),
            pltpu.VMEM((NCHUNK, CHUNK), jnp.int32),
            pltpu.VMEM((NCHUNK, CHUNK), jnp.int32),
            pltpu.VMEM((CHUNK, 128), jnp.float32),
            pltpu.VMEM((CHUNK, 128), jnp.float32),
            pltpu.VMEM((CHUNK, 128), jnp.float32),
            pltpu.VMEM((CHUNK, 128), jnp.float32),
            pltpu.VMEM((BPW,), jnp.float32),
            pltpu.VMEM((BPW,), jnp.float32),
            pltpu.VMEM((BPW,), jnp.float32),
            pltpu.SemaphoreType.DMA,
        ],
    )(u_idx, r_idx, v_idx, E4, Wu4, rv4, bs, bo)


def kernel(u_idx, r_idx, v_idx, E, Wu, rv, bs, bo):
    E4 = _detile(E.T, 65536)
    Wu4 = _detile(Wu.T, 512)
    rv4 = _detile(rv.T, 512)
    return _mure_sc(u_idx, r_idx, v_idx, E4, Wu4, rv4, bs, bo)


# speedup vs baseline: 3.7492x; 1.0658x over previous
import jax
import jax.numpy as jnp
from jax import lax
from jax.experimental import pallas as pl
from jax.experimental.pallas import tpu as pltpu
from jax.experimental.pallas import tpu_sc as plsc

DIM = 32
BATCH = 16384

NC = 2
NS = 16
NW = NC * NS
BPW = BATCH // NW
CHUNK = 64
NCHUNK = BPW // CHUNK
GPC = CHUNK // 16
RROWS = 256


def _detile_body(x0, x1, x2, x3, out_ref):
    q = jnp.concatenate([x0[...], x1[...], x2[...], x3[...]], axis=0)
    out_ref[...] = q.T


def _detile(et, bn):
    n = et.shape[1]
    qb = bn // 4
    last = (n + qb - 1) // qb - 1
    specs = [
        pl.BlockSpec((DIM, qb), lambda i, c=c: (0, jnp.minimum(4 * i + c, last)))
        for c in range(4)
    ]
    grid = (n + bn - 1) // bn
    return pl.pallas_call(
        _detile_body,
        grid=(grid,),
        in_specs=specs,
        out_specs=pl.BlockSpec((qb, 128), lambda i: (i, 0)),
        out_shape=jax.ShapeDtypeStruct((grid * qb, 128), jnp.float32),
    )(et, et, et, et)


def _mure_body(u_idx_hbm, r_idx_hbm, v_idx_hbm, E4_hbm, Wu4_hbm, rv4_hbm,
               bs_hbm, bo_hbm, out_hbm,
               idx_u, idx_r, idx_v, idx4_u, idx4_r, idx4_v,
               u_b0, u_b1, v_b0, v_b1, wu_v, rv_v,
               bs_v, bo_v, out_v, sem0, sem1, rsem, bsem):
    wid = lax.axis_index("s") * NC + lax.axis_index("c")
    base = wid * BPW
    lane = lax.iota(jnp.int32, 16)
    u_bufs, v_bufs, sems = (u_b0, u_b1), (v_b0, v_b1), (sem0, sem1)

    r_cps = [pltpu.async_copy(Wu4_hbm, wu_v, rsem),
             pltpu.async_copy(rv4_hbm, rv_v, rsem)]

    for j in range(NCHUNK):
        sl = pl.ds(base + j * CHUNK, CHUNK)
        pltpu.sync_copy(u_idx_hbm.at[sl], idx_u.at[j])
        pltpu.sync_copy(r_idx_hbm.at[sl], idx_r.at[j])
        pltpu.sync_copy(v_idx_hbm.at[sl], idx_v.at[j])

    def shift(i, carry):
        j = i // GPC
        sl = pl.ds((i % GPC) * 16, 16)
        u = idx_u[j, sl]
        r = idx_r[j, sl]
        v = idx_v[j, sl]
        idx4_u[j, sl] = ((u >> 16) << 14) | (u & 16383)
        idx4_r[j, sl] = ((r >> 9) << 7) | (r & 127)
        idx4_v[j, sl] = ((v >> 16) << 14) | (v & 16383)
        return carry

    lax.fori_loop(0, NCHUNK * GPC, shift, 0)

    bias_cps = []
    for j in range(NCHUNK):
        csl = pl.ds(j * CHUNK, CHUNK)
        bias_cps.append(pltpu.async_copy(bs_hbm.at[idx_u.at[j]], bs_v.at[csl], bsem))
        bias_cps.append(pltpu.async_copy(bo_hbm.at[idx_v.at[j]], bo_v.at[csl], bsem))

    def fire(j):
        b = j % 2
        return [pltpu.async_copy(E4_hbm.at[idx4_u.at[j]], u_bufs[b], sems[b]),
                pltpu.async_copy(E4_hbm.at[idx4_v.at[j]], v_bufs[b], sems[b])]

    pending = {0: fire(0)}
    for cp in r_cps + bias_cps:
        cp.wait()

    for j in range(NCHUNK):
        if j + 1 < NCHUNK:
            pending[j + 1] = fire(j + 1)
        for cp in pending.pop(j):
            cp.wait()
        b = j % 2
        u_rows, v_rows = u_bufs[b], v_bufs[b]

        def group(g, carry):
            sl = pl.ds(g * 16, 16)
            rows16 = g * 16 + lane
            r_rows = idx4_r[j, sl]
            off_u = ((idx_u[j, sl] >> 14) & 3) * DIM
            off_r = ((idx_r[j, sl] >> 7) & 3) * DIM
            off_v = ((idx_v[j, sl] >> 14) & 3) * DIM
            acc = jnp.zeros((16,), jnp.float32)
            for d in range(DIM):
                uu = plsc.load_gather(u_rows, [rows16, off_u + d])
                ru = plsc.load_gather(wu_v, [r_rows, off_r + d])
                vv = plsc.load_gather(v_rows, [rows16, off_v + d])
                rr = plsc.load_gather(rv_v, [r_rows, off_r + d])
                t = uu * ru - vv - rr
                acc = acc + t * t
            osl = pl.ds(j * CHUNK + g * 16, 16)
            out_v[osl] = bs_v[osl] + bo_v[osl] - acc
            return carry

        lax.fori_loop(0, GPC, group, 0)

    pltpu.sync_copy(out_v, out_hbm.at[pl.ds(base, BPW)])


@jax.jit
def _mure_sc(u_idx, r_idx, v_idx, E4, Wu4, rv4, bs, bo):
    mesh = plsc.VectorSubcoreMesh(core_axis_name="c", subcore_axis_name="s")
    return pl.kernel(
        _mure_body,
        mesh=mesh,
        compiler_params=pltpu.CompilerParams(
            needs_layout_passes=False, use_tc_tiling_on_sc=True),
        out_type=jax.ShapeDtypeStruct((BATCH,), jnp.float32),
        scratch_types=[
            pltpu.VMEM((NCHUNK, CHUNK), jnp.int32),
            pltpu.VMEM((NCHUNK, CHUNK), jnp.int32),
            pltpu.VMEM((NCHUNK, CHUNK), jnp.int32),
            pltpu.VMEM((NCHUNK, CHUNK), jnp.int32),
            pltpu.VMEM((NCHUNK, CHUNK), jnp.int32),
            pltpu.VMEM((NCHUNK, CHUNK), jnp.int32),
            pltpu.VMEM((CHUNK, 128), jnp.float32),
            pltpu.VMEM((CHUNK, 128), jnp.float32),
            pltpu.VMEM((CHUNK, 128), jnp.float32),
            pltpu.VMEM((CHUNK, 128), jnp.float32),
            pltpu.VMEM((RROWS, 128), jnp.float32),
            pltpu.VMEM((RROWS, 128), jnp.float32),
            pltpu.VMEM((BPW,), jnp.float32),
            pltpu.VMEM((BPW,), jnp.float32),
            pltpu.VMEM((BPW,), jnp.float32),
            pltpu.SemaphoreType.DMA,
            pltpu.SemaphoreType.DMA,
            pltpu.SemaphoreType.DMA,
            pltpu.SemaphoreType.DMA,
        ],
    )(u_idx, r_idx, v_idx, E4, Wu4, rv4, bs, bo)


def kernel(u_idx, r_idx, v_idx, E, Wu, rv, bs, bo):
    E4 = _detile(E.T, 65536)
    Wu4 = _detile(Wu.T, 512)
    rv4 = _detile(rv.T, 512)
    return _mure_sc(u_idx, r_idx, v_idx, E4, Wu4, rv4, bs, bo)


# speedup vs baseline: 4.2693x; 1.1387x over previous
import jax
import jax.numpy as jnp
from jax import lax
from jax.experimental import pallas as pl
from jax.experimental.pallas import tpu as pltpu
from jax.experimental.pallas import tpu_sc as plsc

DIM = 32
BATCH = 16384

NC = 2
NS = 16
NW = NC * NS
BPW = BATCH // NW
CHUNK = 64
NCHUNK = BPW // CHUNK
GPC = CHUNK // 16
RROWS = 256


def _detile_body(x0, x1, x2, x3, out_ref):
    q = jnp.concatenate([x0[...], x1[...], x2[...], x3[...]], axis=0)
    out_ref[...] = q.T


def _detile(et, bn):
    n = et.shape[1]
    qb = bn // 4
    last = (n + qb - 1) // qb - 1
    specs = [
        pl.BlockSpec((DIM, qb), lambda i, c=c: (0, jnp.minimum(4 * i + c, last)))
        for c in range(4)
    ]
    grid = (n + bn - 1) // bn
    return pl.pallas_call(
        _detile_body,
        grid=(grid,),
        in_specs=specs,
        out_specs=pl.BlockSpec((qb, 128), lambda i: (i, 0)),
        out_shape=jax.ShapeDtypeStruct((grid * qb, 128), jnp.float32),
    )(et, et, et, et)


def _mure_body(u_idx_hbm, r_idx_hbm, v_idx_hbm, E4_hbm, Wu4_hbm, rv4_hbm,
               bs_hbm, bo_hbm, out_hbm,
               idx_u, idx_r, idx_v, idx4_u, idx4_r, idx4_v,
               u_b0, u_b1, v_b0, v_b1, wu_v, rv_v,
               bs_v, bo_v, out_v, sem0, sem1, rsem, bsem):
    wid = lax.axis_index("s") * NC + lax.axis_index("c")
    base = wid * BPW
    lane = lax.iota(jnp.int32, 16)
    idx15 = jnp.full((16,), 15, jnp.int32)
    u_bufs, v_bufs, sems = (u_b0, u_b1), (v_b0, v_b1), (sem0, sem1)

    r_cps = [pltpu.async_copy(Wu4_hbm, wu_v, rsem),
             pltpu.async_copy(rv4_hbm, rv_v, rsem)]

    for j in range(NCHUNK):
        sl = pl.ds(base + j * CHUNK, CHUNK)
        pltpu.sync_copy(u_idx_hbm.at[sl], idx_u.at[j])
        pltpu.sync_copy(r_idx_hbm.at[sl], idx_r.at[j])
        pltpu.sync_copy(v_idx_hbm.at[sl], idx_v.at[j])

    def shift(i, carry):
        j = i // GPC
        sl = pl.ds((i % GPC) * 16, 16)
        u = idx_u[j, sl]
        r = idx_r[j, sl]
        v = idx_v[j, sl]
        idx4_u[j, sl] = ((u >> 16) << 14) | (u & 16383)
        idx4_r[j, sl] = ((r >> 9) << 7) | (r & 127)
        idx4_v[j, sl] = ((v >> 16) << 14) | (v & 16383)
        return carry

    lax.fori_loop(0, NCHUNK * GPC, shift, 0)

    bias_cps = []
    for j in range(NCHUNK):
        csl = pl.ds(j * CHUNK, CHUNK)
        bias_cps.append(pltpu.async_copy(bs_hbm.at[idx_u.at[j]], bs_v.at[csl], bsem))
        bias_cps.append(pltpu.async_copy(bo_hbm.at[idx_v.at[j]], bo_v.at[csl], bsem))

    def fire(j):
        b = j % 2
        return [pltpu.async_copy(E4_hbm.at[idx4_u.at[j]], u_bufs[b], sems[b]),
                pltpu.async_copy(E4_hbm.at[idx4_v.at[j]], v_bufs[b], sems[b])]

    pending = {0: fire(0)}
    for cp in r_cps + bias_cps:
        cp.wait()

    for j in range(NCHUNK):
        if j + 1 < NCHUNK:
            pending[j + 1] = fire(j + 1)
        for cp in pending.pop(j):
            cp.wait()
        b = j % 2
        u_rows, v_rows = u_bufs[b], v_bufs[b]

        def group(g, carry):
            sl = pl.ds(g * 16, 16)
            offu16 = ((idx_u[j, sl] >> 14) & 3) * DIM
            offr16 = ((idx_r[j, sl] >> 7) & 3) * DIM
            offv16 = ((idx_v[j, sl] >> 14) & 3) * DIM
            rrow16 = idx4_r[j, sl]
            score = jnp.zeros((16,), jnp.float32)
            for k in range(16):
                e = g * 16 + k
                off_u = offu16[k]
                off_r = offr16[k]
                off_v = offv16[k]
                rrow = rrow16[k]
                u0 = u_rows[e, pl.ds(off_u, 16)]
                u1 = u_rows[e, pl.ds(off_u + 16, 16)]
                r0 = wu_v[rrow, pl.ds(off_r, 16)]
                r1 = wu_v[rrow, pl.ds(off_r + 16, 16)]
                v0 = v_rows[e, pl.ds(off_v, 16)]
                v1 = v_rows[e, pl.ds(off_v + 16, 16)]
                w0 = rv_v[rrow, pl.ds(off_r, 16)]
                w1 = rv_v[rrow, pl.ds(off_r + 16, 16)]
                t0 = u0 * r0 - v0 - w0
                t1 = u1 * r1 - v1 - w1
                s = plsc.cumsum(t0 * t0 + t1 * t1)
                tot = plsc.cummax(lax.rev(s, dimensions=(0,)))
                score = jnp.where(lane == k, tot, score)
            osl = pl.ds(j * CHUNK + g * 16, 16)
            out_v[osl] = bs_v[osl] + bo_v[osl] - score
            return carry

        lax.fori_loop(0, GPC, group, 0)

    pltpu.sync_copy(out_v, out_hbm.at[pl.ds(base, BPW)])


@jax.jit
def _mure_sc(u_idx, r_idx, v_idx, E4, Wu4, rv4, bs, bo):
    mesh = plsc.VectorSubcoreMesh(core_axis_name="c", subcore_axis_name="s")
    return pl.kernel(
        _mure_body,
        mesh=mesh,
        compiler_params=pltpu.CompilerParams(
            needs_layout_passes=False, use_tc_tiling_on_sc=True),
        out_type=jax.ShapeDtypeStruct((BATCH,), jnp.float32),
        scratch_types=[
            pltpu.VMEM((NCHUNK, CHUNK), jnp.int32),
            pltpu.VMEM((NCHUNK, CHUNK), jnp.int32),
            pltpu.VMEM((NCHUNK, CHUNK), jnp.int32),
            pltpu.VMEM((NCHUNK, CHUNK), jnp.int32),
            pltpu.VMEM((NCHUNK, CHUNK), jnp.int32),
            pltpu.VMEM((NCHUNK, CHUNK), jnp.int32),
            pltpu.VMEM((CHUNK, 128), jnp.float32),
            pltpu.VMEM((CHUNK, 128), jnp.float32),
            pltpu.VMEM((CHUNK, 128), jnp.float32),
            pltpu.VMEM((CHUNK, 128), jnp.float32),
            pltpu.VMEM((RROWS, 128), jnp.float32),
            pltpu.VMEM((RROWS, 128), jnp.float32),
            pltpu.VMEM((BPW,), jnp.float32),
            pltpu.VMEM((BPW,), jnp.float32),
            pltpu.VMEM((BPW,), jnp.float32),
            pltpu.SemaphoreType.DMA,
            pltpu.SemaphoreType.DMA,
            pltpu.SemaphoreType.DMA,
            pltpu.SemaphoreType.DMA,
        ],
    )(u_idx, r_idx, v_idx, E4, Wu4, rv4, bs, bo)


def kernel(u_idx, r_idx, v_idx, E, Wu, rv, bs, bo):
    E4 = _detile(E.T, 65536)
    Wu4 = _detile(Wu.T, 512)
    rv4 = _detile(rv.T, 512)
    return _mure_sc(u_idx, r_idx, v_idx, E4, Wu4, rv4, bs, bo)
